# baseline (speedup 1.0000x reference)
import jax
import jax.numpy as jnp
from jax.experimental import pallas as pl
from jax.experimental.pallas import tpu as pltpu

B, L, D = 2, 2048, 768
H = 12
K = 512
NU = 256
NH = 4
SPLIT = NU // NH
QC = 64
NQ = L // QC


def _body(sas_ref, hs_ref, am_ref, wq_ref, wkv_ref, wo_ref, bo_ref,
          tok_ref, mask_ref, acc_s):
    qi = pl.program_id(1)

    @pl.when(qi == 0)
    def _init():
        acc_s[...] = jnp.zeros_like(acc_s)

    x = sas_ref[0]
    m = x[0]
    for h in range(1, H):
        m = m + x[h]
    m = m * jnp.float32(1.0 / 12.0)
    for t in range(QC // 8):
        acc_s[...] = acc_s[...] + m[8 * t:8 * t + 8]

    @pl.when(qi == NQ - 1)
    def _router():
        acc = acc_s[...]
        a4 = acc[0:4] + acc[4:8]
        a2 = a4[0:2] + a4[2:4]
        imp = a2[0:1] + a2[1:2]

        bits = jax.lax.bitcast_convert_type(imp, jnp.int32)
        key = jnp.where(bits < 0,
                        jnp.bitwise_xor(jnp.bitwise_not(bits),
                                        jnp.int32(-2147483648)),
                        bits)

        def bit_step(i, prefix):
            b = 31 - i
            cand = jnp.where(
                b == 31,
                jnp.int32(0),
                jnp.bitwise_or(prefix, jnp.left_shift(jnp.int32(1), b)))
            cnt = jnp.sum((key >= cand).astype(jnp.int32))
            return jnp.where(cnt >= K, cand, prefix)

        kth = jax.lax.fori_loop(0, 32, bit_step, jnp.int32(-2147483648))

        sel_gt = key > kth
        eq = key == kth
        n_gt = jnp.sum(sel_gt.astype(jnp.int32))
        need_eq = K - n_gt

        lane = jax.lax.broadcasted_iota(jnp.int32, (1, L), 1)

        def excl_cumsum(v):
            s = v
            k = 1
            while k < L:
                sh = pltpu.roll(s, k, 1)
                s = s + jnp.where(lane >= k, sh, jnp.int32(0))
                k *= 2
            return s - v

        rank_eq = excl_cumsum(eq.astype(jnp.int32))
        sel = jnp.logical_or(sel_gt,
                             jnp.logical_and(eq, rank_eq < need_eq))
        rank_i = excl_cumsum(sel.astype(jnp.int32))

        kio = jax.lax.broadcasted_iota(jnp.int32, (K, L), 0)
        pmat = jnp.where(
            jnp.logical_and(jnp.broadcast_to(sel, (K, L)),
                            jnp.broadcast_to(rank_i, (K, L)) == kio),
            jnp.float32(1.0), jnp.float32(0.0))

        hs = hs_ref[0]
        preserved = jax.lax.dot_general(
            pmat, hs, (((1,), (0,)), ((), ())),
            preferred_element_type=jnp.float32)

        am = am_ref[0]
        pam = jax.lax.dot_general(
            am, pmat, (((1,), (1,)), ((), ())),
            preferred_element_type=jnp.float32)

        mx = jnp.max(am)
        e = jnp.exp(am - mx)
        att = e / jnp.sum(e)
        sentences = jax.lax.dot_general(
            att, hs, (((1,), (0,)), ((), ())),
            preferred_element_type=jnp.float32)

        q_row = jax.lax.dot_general(sentences, wq_ref[...],
                                    (((1,), (0,)), ((), ())),
                                    preferred_element_type=jnp.float32)
        kvmat = jax.lax.dot_general(hs, wkv_ref[...],
                                    (((1,), (0,)), ((), ())),
                                    preferred_element_type=jnp.float32)

        kpm = am < jnp.float32(-10.0)
        scale = jnp.float32(1.0 / (768.0 ** 0.5))
        heads = []
        for h in range(NH):
            qh = q_row[:, h * SPLIT:(h + 1) * SPLIT]
            kh = kvmat[:, h * SPLIT:(h + 1) * SPLIT]
            vh = kvmat[:, NU + h * SPLIT:NU + (h + 1) * SPLIT]
            s = jax.lax.dot_general(qh, kh, (((1,), (1,)), ((), ())),
                                    preferred_element_type=jnp.float32)
            s = s * scale
            s = jnp.where(kpm, -jnp.inf, s)
            smx = jnp.max(s)
            se = jnp.exp(s - smx)
            p = se / jnp.sum(se)
            oh = jax.lax.dot_general(p, vh, (((1,), (0,)), ((), ())),
                                     preferred_element_type=jnp.float32)
            heads.append(oh)
        o = jnp.concatenate(heads, axis=1)
        new_tok = jax.lax.dot_general(o, wo_ref[...],
                                      (((1,), (0,)), ((), ())),
                                      preferred_element_type=jnp.float32)
        new_tok = new_tok + bo_ref[...]

        tok_ref[0] = jnp.concatenate([hs[0:1, :], preserved, new_tok],
                                     axis=0)
        zero1 = jnp.zeros((1, 1), jnp.float32)
        mask_ref[0, 0] = jnp.concatenate([zero1, pam, zero1], axis=1)


def kernel(hidden_states, attention_mask, self_attention_scores,
           Wq, Wk, Wv, Wo, bo):
    am3 = attention_mask.reshape(B, 1, L)
    wkv = jnp.concatenate([Wk, Wv], axis=1)
    bo2 = bo.reshape(1, D)
    tok, msk = pl.pallas_call(
        _body,
        grid=(B, NQ),
        in_specs=[
            pl.BlockSpec((1, H, QC, L), lambda b, q: (b, 0, q, 0)),
            pl.BlockSpec((1, L, D), lambda b, q: (b, 0, 0)),
            pl.BlockSpec((1, 1, L), lambda b, q: (b, 0, 0)),
            pl.BlockSpec((D, NU), lambda b, q: (0, 0)),
            pl.BlockSpec((D, 2 * NU), lambda b, q: (0, 0)),
            pl.BlockSpec((NU, D), lambda b, q: (0, 0)),
            pl.BlockSpec((1, D), lambda b, q: (0, 0)),
        ],
        out_specs=[
            pl.BlockSpec((1, K + 2, D), lambda b, q: (b, 0, 0)),
            pl.BlockSpec((1, 1, 1, K + 2), lambda b, q: (b, 0, 0, 0)),
        ],
        out_shape=[
            jax.ShapeDtypeStruct((B, K + 2, D), jnp.float32),
            jax.ShapeDtypeStruct((B, 1, 1, K + 2), jnp.float32),
        ],
        scratch_shapes=[pltpu.VMEM((8, L), jnp.float32)],
        compiler_params=pltpu.CompilerParams(
            dimension_semantics=("arbitrary", "arbitrary"),
        ),
    )(self_attention_scores, hidden_states, am3, Wq, wkv, Wo, bo2)
    return (tok, msk)


# speedup vs baseline: 1.0296x; 1.0296x over previous
import jax
import jax.numpy as jnp
from jax.experimental import pallas as pl
from jax.experimental.pallas import tpu as pltpu

B, L, D = 2, 2048, 768
H = 12
K = 512
NU = 256
NH = 4
SPLIT = NU // NH
QC = 128
NQ = L // QC


def _body(sas_ref, hs_ref, am_ref, wq_ref, wkv_ref, wo_ref, bo_ref,
          tok_ref, mask_ref, acc_s, new_s):
    qi = pl.program_id(1)

    @pl.when(qi == 0)
    def _init():
        acc_s[...] = jnp.zeros_like(acc_s)

    @pl.when(qi == 1)
    def _mha():
        hs = hs_ref[0]
        am = am_ref[0]
        mx = jnp.max(am)
        e = jnp.exp(am - mx)
        att = e / jnp.sum(e)
        sentences = jax.lax.dot_general(
            att, hs, (((1,), (0,)), ((), ())),
            preferred_element_type=jnp.float32)

        q_row = jax.lax.dot_general(sentences, wq_ref[...],
                                    (((1,), (0,)), ((), ())),
                                    preferred_element_type=jnp.float32)
        kvmat = jax.lax.dot_general(hs, wkv_ref[...],
                                    (((1,), (0,)), ((), ())),
                                    preferred_element_type=jnp.float32)

        kpm = am < jnp.float32(-10.0)
        scale = jnp.float32(1.0 / (768.0 ** 0.5))
        heads = []
        for h in range(NH):
            qh = q_row[:, h * SPLIT:(h + 1) * SPLIT]
            kh = kvmat[:, h * SPLIT:(h + 1) * SPLIT]
            vh = kvmat[:, NU + h * SPLIT:NU + (h + 1) * SPLIT]
            s = jax.lax.dot_general(qh, kh, (((1,), (1,)), ((), ())),
                                    preferred_element_type=jnp.float32)
            s = s * scale
            s = jnp.where(kpm, -jnp.inf, s)
            smx = jnp.max(s)
            se = jnp.exp(s - smx)
            p = se / jnp.sum(se)
            oh = jax.lax.dot_general(p, vh, (((1,), (0,)), ((), ())),
                                     preferred_element_type=jnp.float32)
            heads.append(oh)
        o = jnp.concatenate(heads, axis=1)
        new_tok = jax.lax.dot_general(o, wo_ref[...],
                                      (((1,), (0,)), ((), ())),
                                      preferred_element_type=jnp.float32)
        new_s[...] = new_tok + bo_ref[...]

    x = sas_ref[0]
    m = x[0]
    for h in range(1, H):
        m = m + x[h]
    m = m * jnp.float32(1.0 / 12.0)
    for t in range(QC // 8):
        acc_s[...] = acc_s[...] + m[8 * t:8 * t + 8]

    @pl.when(qi == NQ - 1)
    def _router():
        acc = acc_s[...]
        a4 = acc[0:4] + acc[4:8]
        a2 = a4[0:2] + a4[2:4]
        imp = a2[0:1] + a2[1:2]

        bits = jax.lax.bitcast_convert_type(imp, jnp.int32)
        key = jnp.where(bits < 0,
                        jnp.bitwise_xor(jnp.bitwise_not(bits),
                                        jnp.int32(-2147483648)),
                        bits)

        def bit_step(i, prefix):
            b = 31 - i
            cand = jnp.where(
                b == 31,
                jnp.int32(0),
                jnp.bitwise_or(prefix, jnp.left_shift(jnp.int32(1), b)))
            cnt = jnp.sum((key >= cand).astype(jnp.int32))
            return jnp.where(cnt >= K, cand, prefix)

        kth = jax.lax.fori_loop(0, 32, bit_step, jnp.int32(-2147483648))

        sel_gt = key > kth
        eq = key == kth
        n_gt = jnp.sum(sel_gt.astype(jnp.int32))
        need_eq = K - n_gt

        lane = jax.lax.broadcasted_iota(jnp.int32, (1, L), 1)

        def excl_cumsum(v):
            s = v
            k = 1
            while k < L:
                sh = pltpu.roll(s, k, 1)
                s = s + jnp.where(lane >= k, sh, jnp.int32(0))
                k *= 2
            return s - v

        rank_eq = excl_cumsum(eq.astype(jnp.int32))
        sel = jnp.logical_or(sel_gt,
                             jnp.logical_and(eq, rank_eq < need_eq))
        rank_i = excl_cumsum(sel.astype(jnp.int32))

        kio = jax.lax.broadcasted_iota(jnp.int32, (K, L), 0)
        pmat = jnp.where(
            jnp.logical_and(jnp.broadcast_to(sel, (K, L)),
                            jnp.broadcast_to(rank_i, (K, L)) == kio),
            jnp.float32(1.0), jnp.float32(0.0))

        hs = hs_ref[0]
        preserved = jax.lax.dot_general(
            pmat, hs, (((1,), (0,)), ((), ())),
            preferred_element_type=jnp.float32)

        am = am_ref[0]
        pam = jax.lax.dot_general(
            am, pmat, (((1,), (1,)), ((), ())),
            preferred_element_type=jnp.float32)

        tok_ref[0] = jnp.concatenate([hs[0:1, :], preserved, new_s[...]],
                                     axis=0)
        zero1 = jnp.zeros((1, 1), jnp.float32)
        mask_ref[0, 0] = jnp.concatenate([zero1, pam, zero1], axis=1)


def kernel(hidden_states, attention_mask, self_attention_scores,
           Wq, Wk, Wv, Wo, bo):
    am3 = attention_mask.reshape(B, 1, L)
    wkv = jnp.concatenate([Wk, Wv], axis=1)
    bo2 = bo.reshape(1, D)
    tok, msk = pl.pallas_call(
        _body,
        grid=(B, NQ),
        in_specs=[
            pl.BlockSpec((1, H, QC, L), lambda b, q: (b, 0, q, 0)),
            pl.BlockSpec((1, L, D), lambda b, q: (b, 0, 0)),
            pl.BlockSpec((1, 1, L), lambda b, q: (b, 0, 0)),
            pl.BlockSpec((D, NU), lambda b, q: (0, 0)),
            pl.BlockSpec((D, 2 * NU), lambda b, q: (0, 0)),
            pl.BlockSpec((NU, D), lambda b, q: (0, 0)),
            pl.BlockSpec((1, D), lambda b, q: (0, 0)),
        ],
        out_specs=[
            pl.BlockSpec((1, K + 2, D), lambda b, q: (b, 0, 0)),
            pl.BlockSpec((1, 1, 1, K + 2), lambda b, q: (b, 0, 0, 0)),
        ],
        out_shape=[
            jax.ShapeDtypeStruct((B, K + 2, D), jnp.float32),
            jax.ShapeDtypeStruct((B, 1, 1, K + 2), jnp.float32),
        ],
        scratch_shapes=[pltpu.VMEM((8, L), jnp.float32),
                        pltpu.VMEM((1, D), jnp.float32)],
        compiler_params=pltpu.CompilerParams(
            dimension_semantics=("arbitrary", "arbitrary"),
        ),
    )(self_attention_scores, hidden_states, am3, Wq, wkv, Wo, bo2)
    return (tok, msk)
